# baseline (speedup 1.0000x reference)
import functools

import jax
import jax.numpy as jnp
from jax import lax
from jax.experimental import pallas as pl
from jax.experimental.pallas import tpu as pltpu
from jax.experimental.pallas import tpu_sc as plsc

_SCALE = 8.0
_EPS = 1e-5
_GATHER_W = 40
_POS_BLK = 128
_LN_BATCH = 16


def _sc_gather2(tok_table, pos_table, tokens, positions):
    b, s = tokens.shape
    d = tok_table.shape[1]
    n = b * s
    nw = s // _GATHER_W
    mesh = plsc.VectorSubcoreMesh(core_axis_name="c", subcore_axis_name="s")
    row_type = jax.ShapeDtypeStruct((n, d), tok_table.dtype)

    @functools.partial(
        pl.kernel,
        out_type=[row_type, row_type],
        mesh=mesh,
        compiler_params=pltpu.CompilerParams(use_tc_tiling_on_sc=False),
    )
    def gather_k(tab_hbm, ptab_hbm, tok_hbm, pos_hbm, tout_hbm, pout_hbm):
        def body(ti_vmem, pi_vmem, to_vmem, po_vmem):
            pltpu.sync_copy(tab_hbm.at[ti_vmem.at[0]], to_vmem)
            pltpu.sync_copy(ptab_hbm.at[pi_vmem.at[0]], po_vmem)

        pltpu.emit_pipeline(
            body,
            grid=(b, nw),
            in_specs=[
                pl.BlockSpec((1, _GATHER_W), lambda i, j: (i, j)),
                pl.BlockSpec((1, _GATHER_W), lambda i, j: (i, j)),
            ],
            out_specs=[
                pl.BlockSpec((_GATHER_W, d), lambda i, j, _nw=nw: (i * _nw + j, 0)),
                pl.BlockSpec((_GATHER_W, d), lambda i, j, _nw=nw: (i * _nw + j, 0)),
            ],
            core_axis_name=("c", "s"),
            dimension_semantics=(pltpu.PARALLEL, pltpu.PARALLEL),
        )(tok_hbm, pos_hbm, tout_hbm, pout_hbm)

    return gather_k(tok_table, pos_table, tokens, positions)


def _positions_body(tok_ref, pos_ref):
    tok = tok_ref[...]
    mask = tok != 0
    mb = mask.astype(jnp.bfloat16)
    s = tok.shape[1]
    r = lax.broadcasted_iota(jnp.int32, (s, s), 0)
    c = lax.broadcasted_iota(jnp.int32, (s, s), 1)
    tri = (r <= c).astype(jnp.bfloat16)
    cs = jnp.dot(mb, tri, preferred_element_type=jnp.float32)
    pos_ref[...] = cs.astype(jnp.int32) * mask.astype(jnp.int32)


def _ln_body(tok_ref, pos_ref, gamma_ref, beta_ref, out_ref):
    x = tok_ref[...] * _SCALE + pos_ref[...]
    mean = jnp.mean(x, axis=1, keepdims=True)
    xc = x - mean
    var = jnp.mean(xc * xc, axis=1, keepdims=True)
    inv = lax.rsqrt(var + _EPS)
    y = xc * inv * gamma_ref[...] + beta_ref[...]
    blk, s, d = out_ref.shape
    out_ref[...] = y.reshape(blk, s, d)


def kernel(tokens, tok_table, pos_table, gamma, beta):
    b, s = tokens.shape
    d = tok_table.shape[1]
    n = b * s
    tokens = tokens.astype(jnp.int32)

    pos = pl.pallas_call(
        _positions_body,
        grid=(b // _POS_BLK,),
        in_specs=[pl.BlockSpec((_POS_BLK, s), lambda i: (i, 0))],
        out_specs=pl.BlockSpec((_POS_BLK, s), lambda i: (i, 0)),
        out_shape=jax.ShapeDtypeStruct((b, s), jnp.int32),
    )(tokens)

    tok_emb, pos_emb = _sc_gather2(tok_table, pos_table, tokens, pos)

    rows = _LN_BATCH * s
    out = pl.pallas_call(
        _ln_body,
        grid=(b // _LN_BATCH,),
        in_specs=[
            pl.BlockSpec((rows, d), lambda i: (i, 0)),
            pl.BlockSpec((rows, d), lambda i: (i, 0)),
            pl.BlockSpec((1, d), lambda i: (0, 0)),
            pl.BlockSpec((1, d), lambda i: (0, 0)),
        ],
        out_specs=pl.BlockSpec((_LN_BATCH, s, d), lambda i: (i, 0, 0)),
        out_shape=jax.ShapeDtypeStruct((b, s, d), jnp.float32),
    )(tok_emb, pos_emb, gamma.reshape(1, d), beta.reshape(1, d))

    return out


# speedup vs baseline: 1.2943x; 1.2943x over previous
import functools

import jax
import jax.numpy as jnp
from jax import lax
from jax.experimental import pallas as pl
from jax.experimental.pallas import tpu as pltpu
from jax.experimental.pallas import tpu_sc as plsc

_SCALE = 8.0
_EPS = 1e-5
_POS_PAD = 256
_GATHER_W = 128
_LN_BATCH = 16


def _sc_gather(table, idx):
    n = idx.shape[1]
    d = table.shape[1]
    mesh = plsc.VectorSubcoreMesh(core_axis_name="c", subcore_axis_name="s")

    @functools.partial(
        pl.kernel,
        out_type=jax.ShapeDtypeStruct((n, d), table.dtype),
        mesh=mesh,
        compiler_params=pltpu.CompilerParams(use_tc_tiling_on_sc=False),
    )
    def gather_k(tab_hbm, idx_hbm, out_hbm):
        def body(i_vmem, o_vmem):
            pltpu.sync_copy(tab_hbm.at[i_vmem.at[0]], o_vmem)

        pltpu.emit_pipeline(
            body,
            grid=(n // _GATHER_W,),
            in_specs=[pl.BlockSpec((1, _GATHER_W), lambda i: (0, i))],
            out_specs=[pl.BlockSpec((_GATHER_W, d), lambda i: (i, 0))],
            core_axis_name=("c", "s"),
            dimension_semantics=(pltpu.PARALLEL,),
        )(idx_hbm, out_hbm)

    return gather_k(table, idx)


def _ln_body(tokens_ref, tok_ref, hi_ref, lo_ref, gamma_ref, beta_ref, out_ref):
    bb, s, d = out_ref.shape
    tok = tokens_ref[...]
    maskf = (tok != 0).astype(jnp.float32)
    mask_t = maskf.T
    r = lax.broadcasted_iota(jnp.int32, (s, s), 0)
    c = lax.broadcasted_iota(jnp.int32, (s, s), 1)
    tri_low = (r >= c).astype(jnp.bfloat16)
    pos_t = jnp.dot(tri_low, mask_t.astype(jnp.bfloat16),
                    preferred_element_type=jnp.float32) * mask_t
    iota_f = lax.broadcasted_iota(jnp.int32, (1, _POS_PAD), 1).astype(jnp.float32)
    hi = hi_ref[...]
    lo = lo_ref[...]
    pes = []
    for b in range(bb):
        oh = (pos_t[:, b : b + 1] == iota_f).astype(jnp.bfloat16)
        pe_b = jnp.dot(oh, hi, preferred_element_type=jnp.float32)
        pe_b = pe_b + jnp.dot(oh, lo, preferred_element_type=jnp.float32)
        pes.append(pe_b)
    pe = jnp.stack(pes, axis=0)
    x = tok_ref[...].reshape(bb, s, d) * _SCALE + pe
    mean = jnp.mean(x, axis=2, keepdims=True)
    xc = x - mean
    var = jnp.mean(xc * xc, axis=2, keepdims=True)
    inv = lax.rsqrt(var + _EPS)
    gamma = gamma_ref[...].reshape(1, 1, d)
    beta = beta_ref[...].reshape(1, 1, d)
    out_ref[...] = xc * inv * gamma + beta


def kernel(tokens, tok_table, pos_table, gamma, beta):
    b, s = tokens.shape
    d = tok_table.shape[1]
    n = b * s
    tokens = tokens.astype(jnp.int32)

    tok_emb = _sc_gather(tok_table, tokens.reshape(1, n))

    pt = jnp.zeros((_POS_PAD, d), jnp.float32).at[: pos_table.shape[0]].set(pos_table)
    hi = pt.astype(jnp.bfloat16)
    lo = (pt - hi.astype(jnp.float32)).astype(jnp.bfloat16)

    rows = _LN_BATCH * s
    out = pl.pallas_call(
        _ln_body,
        grid=(b // _LN_BATCH,),
        in_specs=[
            pl.BlockSpec((_LN_BATCH, s), lambda i: (i, 0)),
            pl.BlockSpec((rows, d), lambda i: (i, 0)),
            pl.BlockSpec((_POS_PAD, d), lambda i: (0, 0)),
            pl.BlockSpec((_POS_PAD, d), lambda i: (0, 0)),
            pl.BlockSpec((1, d), lambda i: (0, 0)),
            pl.BlockSpec((1, d), lambda i: (0, 0)),
        ],
        out_specs=pl.BlockSpec((_LN_BATCH, s, d), lambda i: (i, 0, 0)),
        out_shape=jax.ShapeDtypeStruct((b, s, d), jnp.float32),
    )(tokens, tok_emb, hi, lo, gamma.reshape(1, d), beta.reshape(1, d))

    return out


# speedup vs baseline: 1.4656x; 1.1324x over previous
import functools

import jax
import jax.numpy as jnp
from jax import lax
from jax.experimental import pallas as pl
from jax.experimental.pallas import tpu as pltpu
from jax.experimental.pallas import tpu_sc as plsc

_SCALE = 8.0
_EPS = 1e-5
_POS_PAD = 256
_GATHER_W = 128
_LN_BATCH = 16


def _sc_gather(table, idx):
    n = idx.shape[1]
    d = table.shape[1]
    mesh = plsc.VectorSubcoreMesh(core_axis_name="c", subcore_axis_name="s")

    @functools.partial(
        pl.kernel,
        out_type=jax.ShapeDtypeStruct((n, d), table.dtype),
        mesh=mesh,
    )
    def gather_k(tab_hbm, idx_hbm, out_hbm):
        def body(i_vmem, o_vmem):
            pltpu.sync_copy(tab_hbm.at[i_vmem.at[0]], o_vmem)

        pltpu.emit_pipeline(
            body,
            grid=(n // _GATHER_W,),
            in_specs=[pl.BlockSpec((1, _GATHER_W), lambda i: (0, i))],
            out_specs=[pl.BlockSpec((_GATHER_W, d), lambda i: (i, 0))],
            core_axis_name=("c", "s"),
            dimension_semantics=(pltpu.PARALLEL,),
        )(idx_hbm, out_hbm)

    return gather_k(table, idx)


def _ln_body(tokens_ref, tok_ref, hi_ref, lo_ref, gamma_ref, beta_ref, out_ref):
    bb, s, d = out_ref.shape
    dp = tok_ref.shape[1]
    tok = tokens_ref[...]
    maskf = (tok != 0).astype(jnp.float32)
    mask_t = maskf.T
    r = lax.broadcasted_iota(jnp.int32, (s, s), 0)
    c = lax.broadcasted_iota(jnp.int32, (s, s), 1)
    tri_low = (r >= c).astype(jnp.bfloat16)
    pos_t = jnp.dot(tri_low, mask_t.astype(jnp.bfloat16),
                    preferred_element_type=jnp.float32) * mask_t
    iota_f = lax.broadcasted_iota(jnp.int32, (1, _POS_PAD), 1).astype(jnp.float32)
    hi = hi_ref[...]
    lo = lo_ref[...]
    pes = []
    for b in range(bb):
        oh = (pos_t[:, b : b + 1] == iota_f).astype(jnp.bfloat16)
        pe_b = jnp.dot(oh, hi, preferred_element_type=jnp.float32)
        pe_b = pe_b + jnp.dot(oh, lo, preferred_element_type=jnp.float32)
        pes.append(pe_b)
    pe = jnp.stack(pes, axis=0)
    x = tok_ref[...].reshape(bb, s, dp) * _SCALE + pe
    mean = jnp.sum(x, axis=2, keepdims=True) * (1.0 / d)
    sumsq = jnp.sum(x * x, axis=2, keepdims=True) * (1.0 / d)
    var = sumsq - mean * mean
    inv = lax.rsqrt(var + _EPS)
    y = ((x - mean) * inv)[:, :, :d]
    gamma = gamma_ref[...].reshape(1, 1, d)
    beta = beta_ref[...].reshape(1, 1, d)
    out_ref[...] = y * gamma + beta


def kernel(tokens, tok_table, pos_table, gamma, beta):
    b, s = tokens.shape
    d = tok_table.shape[1]
    n = b * s
    tokens = tokens.astype(jnp.int32)
    dp = 2 * d

    tab128 = jnp.pad(tok_table, ((0, 0), (0, dp - d)))
    tok_emb = _sc_gather(tab128, tokens.reshape(1, n))

    pt = jnp.zeros((_POS_PAD, dp), jnp.float32).at[: pos_table.shape[0], :d].set(pos_table)
    hi = pt.astype(jnp.bfloat16)
    lo = (pt - hi.astype(jnp.float32)).astype(jnp.bfloat16)

    rows = _LN_BATCH * s
    out = pl.pallas_call(
        _ln_body,
        grid=(b // _LN_BATCH,),
        in_specs=[
            pl.BlockSpec((_LN_BATCH, s), lambda i: (i, 0)),
            pl.BlockSpec((rows, dp), lambda i: (i, 0)),
            pl.BlockSpec((_POS_PAD, dp), lambda i: (0, 0)),
            pl.BlockSpec((_POS_PAD, dp), lambda i: (0, 0)),
            pl.BlockSpec((1, d), lambda i: (0, 0)),
            pl.BlockSpec((1, d), lambda i: (0, 0)),
        ],
        out_specs=pl.BlockSpec((_LN_BATCH, s, d), lambda i: (i, 0, 0)),
        out_shape=jax.ShapeDtypeStruct((b, s, d), jnp.float32),
    )(tokens, tok_emb, hi, lo, gamma.reshape(1, d), beta.reshape(1, d))

    return out


# speedup vs baseline: 2.1328x; 1.4552x over previous
import functools

import jax
import jax.numpy as jnp
from jax import lax
from jax.experimental import pallas as pl
from jax.experimental.pallas import tpu as pltpu
from jax.experimental.pallas import tpu_sc as plsc

_SCALE = 8.0
_EPS = 1e-5
_POS_PAD = 256
_GATHER_W = 128
_LN_BATCH = 16


def _sc_gather(table, idx):
    n = idx.shape[1]
    d = table.shape[1]
    mesh = plsc.VectorSubcoreMesh(core_axis_name="c", subcore_axis_name="s")

    @functools.partial(
        pl.kernel,
        out_type=jax.ShapeDtypeStruct((n, d), table.dtype),
        mesh=mesh,
    )
    def gather_k(tab_hbm, idx_hbm, out_hbm):
        def body(i_vmem, o_vmem):
            pltpu.sync_copy(tab_hbm.at[i_vmem.at[0]], o_vmem)

        pltpu.emit_pipeline(
            body,
            grid=(n // _GATHER_W,),
            in_specs=[pl.BlockSpec((1, _GATHER_W), lambda i: (0, i))],
            out_specs=[pl.BlockSpec((_GATHER_W, d), lambda i: (i, 0))],
            core_axis_name=("c", "s"),
            dimension_semantics=(pltpu.PARALLEL,),
        )(idx_hbm, out_hbm)

    return gather_k(table, idx)


def _pad_body(tabt_ref, out_ref):
    x = tabt_ref[...]
    c, dp = out_ref.shape
    d = x.shape[0]
    y = jnp.transpose(x)
    out_ref[:, :d] = y
    out_ref[:, d:] = jnp.zeros((c, dp - d), x.dtype)


def _transpose_pad(table):
    v, d = table.shape
    c = 8192
    grid = (v + c - 1) // c
    return pl.pallas_call(
        _pad_body,
        grid=(grid,),
        in_specs=[pl.BlockSpec((d, c), lambda i: (0, i))],
        out_specs=pl.BlockSpec((c, 2 * d), lambda i: (i, 0)),
        out_shape=jax.ShapeDtypeStruct((v, 2 * d), table.dtype),
        compiler_params=pltpu.CompilerParams(dimension_semantics=("parallel",)),
    )(table.T)


def _ln_body(tokens_ref, tok_ref, hi_ref, lo_ref, gamma_ref, beta_ref, out_ref):
    bb, s, d = out_ref.shape
    dp = tok_ref.shape[1]
    tok = tokens_ref[...]
    maskf = (tok != 0).astype(jnp.float32)
    mask_t = maskf.T
    r = lax.broadcasted_iota(jnp.int32, (s, s), 0)
    c = lax.broadcasted_iota(jnp.int32, (s, s), 1)
    tri_low = (r >= c).astype(jnp.bfloat16)
    pos_t = jnp.dot(tri_low, mask_t.astype(jnp.bfloat16),
                    preferred_element_type=jnp.float32) * mask_t
    iota_f = lax.broadcasted_iota(jnp.int32, (1, _POS_PAD), 1).astype(jnp.float32)
    hi = hi_ref[...]
    lo = lo_ref[...]
    pes = []
    for b in range(bb):
        oh = (pos_t[:, b : b + 1] == iota_f).astype(jnp.bfloat16)
        pe_b = jnp.dot(oh, hi, preferred_element_type=jnp.float32)
        pe_b = pe_b + jnp.dot(oh, lo, preferred_element_type=jnp.float32)
        pes.append(pe_b)
    pe = jnp.stack(pes, axis=0)
    x = tok_ref[...].reshape(bb, s, dp) * _SCALE + pe
    mean = jnp.sum(x, axis=2, keepdims=True) * (1.0 / d)
    sumsq = jnp.sum(x * x, axis=2, keepdims=True) * (1.0 / d)
    var = sumsq - mean * mean
    inv = lax.rsqrt(var + _EPS)
    y = ((x - mean) * inv)[:, :, :d]
    gamma = gamma_ref[...].reshape(1, 1, d)
    beta = beta_ref[...].reshape(1, 1, d)
    out_ref[...] = y * gamma + beta


def kernel(tokens, tok_table, pos_table, gamma, beta):
    b, s = tokens.shape
    d = tok_table.shape[1]
    n = b * s
    tokens = tokens.astype(jnp.int32)
    dp = 2 * d

    tab128 = _transpose_pad(tok_table)
    tok_emb = _sc_gather(tab128, tokens.reshape(1, n))

    pt = jnp.zeros((_POS_PAD, dp), jnp.float32).at[: pos_table.shape[0], :d].set(pos_table)
    hi = pt.astype(jnp.bfloat16)
    lo = (pt - hi.astype(jnp.float32)).astype(jnp.bfloat16)

    rows = _LN_BATCH * s
    out = pl.pallas_call(
        _ln_body,
        grid=(b // _LN_BATCH,),
        in_specs=[
            pl.BlockSpec((_LN_BATCH, s), lambda i: (i, 0)),
            pl.BlockSpec((rows, dp), lambda i: (i, 0)),
            pl.BlockSpec((_POS_PAD, dp), lambda i: (0, 0)),
            pl.BlockSpec((_POS_PAD, dp), lambda i: (0, 0)),
            pl.BlockSpec((1, d), lambda i: (0, 0)),
            pl.BlockSpec((1, d), lambda i: (0, 0)),
        ],
        out_specs=pl.BlockSpec((_LN_BATCH, s, d), lambda i: (i, 0, 0)),
        out_shape=jax.ShapeDtypeStruct((b, s, d), jnp.float32),
        compiler_params=pltpu.CompilerParams(dimension_semantics=("parallel",)),
    )(tokens, tok_emb, hi, lo, gamma.reshape(1, d), beta.reshape(1, d))

    return out
